# baseline (speedup 1.0000x reference)
import functools

import jax
import jax.numpy as jnp
from jax.experimental import pallas as pl

DQ = 128
DM = 512


def _tc_full(idx_ref, ct_ref, p_ref, sw_ref, sb_ref, w_ref, b_ref,
             o_ref, *, h, w):
    n = h * w
    nc = ct_ref.shape[0]
    wc = w_ref[0:DQ, :]
    wp = w_ref[DQ:2 * DQ, :]
    ws = w_ref[2 * DQ:3 * DQ, :]
    size_e = float(h) * sw_ref[0:1, :] + float(w) * sw_ref[1:2, :] + sb_ref[0:1, :]
    const = jnp.dot(size_e, ws, preferred_element_type=jnp.float32) + b_ref[0:1, :]
    zt = jnp.dot(ct_ref[...], wc, preferred_element_type=jnp.float32)
    zt = jnp.concatenate([zt, jnp.zeros((DQ - nc, DM), jnp.float32)], axis=0)
    lanes = jax.lax.broadcasted_iota(jnp.int32, (n, DQ), 1)
    oh = (lanes == idx_ref[...]).astype(jnp.float32)
    acc = jnp.dot(oh, zt, preferred_element_type=jnp.float32)
    acc = acc + jnp.dot(p_ref[...], wp, preferred_element_type=jnp.float32)
    o_ref[...] = acc + const


def kernel(grid, color_table, pos_emb, size_W, size_b, combine_W, combine_b):
    h, w = grid.shape
    n = h * w
    out = pl.pallas_call(
        functools.partial(_tc_full, h=h, w=w),
        out_shape=jax.ShapeDtypeStruct((n, DM), jnp.float32),
    )(
        grid.reshape(n, 1).astype(jnp.int32),
        color_table,
        pos_emb[:h, :w].reshape(n, DQ),
        size_W,
        size_b.reshape(1, DQ),
        combine_W,
        combine_b.reshape(1, DM),
    )
    return out.reshape(h, w, DM)


# speedup vs baseline: 2.6911x; 2.6911x over previous
import functools

import jax
import jax.numpy as jnp
from jax.experimental import pallas as pl

DQ = 128
DM = 512


def _tc_full(idx_ref, ct_ref, p_ref, sw_ref, sb_ref, w_ref, b_ref,
             o_ref, *, h, w):
    nc = ct_ref.shape[0]
    wc = w_ref[0:DQ, :]
    wp = w_ref[DQ:2 * DQ, :]
    ws = w_ref[2 * DQ:3 * DQ, :]
    size_e = float(h) * sw_ref[0:1, :] + float(w) * sw_ref[1:2, :] + sb_ref[0:1, :]
    const = jnp.dot(size_e, ws, preferred_element_type=jnp.float32) + b_ref[0:1, :]
    zt = jnp.dot(ct_ref[...], wc, preferred_element_type=jnp.float32)
    zt = jnp.concatenate([zt, jnp.zeros((DQ - nc, DM), jnp.float32)], axis=0)
    lanes = jax.lax.broadcasted_iota(jnp.int32, (h, w, DQ), 2)
    oh = (lanes == idx_ref[...][:, :, None]).astype(jnp.float32)
    dn = (((2,), (0,)), ((), ()))
    acc = jax.lax.dot_general(oh, zt, dn, preferred_element_type=jnp.float32)
    acc = acc + jax.lax.dot_general(p_ref[...], wp, dn,
                                    preferred_element_type=jnp.float32)
    o_ref[...] = acc + const.reshape(1, 1, DM)


def kernel(grid, color_table, pos_emb, size_W, size_b, combine_W, combine_b):
    h, w = grid.shape
    return pl.pallas_call(
        functools.partial(_tc_full, h=h, w=w),
        out_shape=jax.ShapeDtypeStruct((h, w, DM), jnp.float32),
    )(
        grid.astype(jnp.int32),
        color_table,
        pos_emb[:h, :w],
        size_W,
        size_b.reshape(1, DQ),
        combine_W,
        combine_b.reshape(1, DM),
    )
